# baseline (speedup 1.0000x reference)
import functools

import jax
import jax.numpy as jnp
from jax import lax
from jax.experimental import pallas as pl
from jax.experimental.pallas import tpu as pltpu
from jax.experimental.pallas import tpu_sc as plsc

_K = 8
_L = 16
_N = 8192
_B = 32
_R = 32
_NW = 32
_RPW = (_B * _R) // _NW
_GSZ = _L * _L
_NG = _N // _GSZ
_NEG = float("-inf")


_NBUF = 8


_SPLIT = 4
_RPW_H = _R - _SPLIT


def _sc_body(x_hbm, out_hbm, buf, bm, cand, outb,
             sem0, sem1, sem2, sem3, sem4, sem5, sem6, sem7):
    w = lax.axis_index("s") * 2 + lax.axis_index("c")
    row0 = _SPLIT * _R + w * _RPW_H
    iota = lax.iota(jnp.int32, _L)
    neg = jnp.full((_L,), _NEG, jnp.float32)

    sems = (sem0, sem1, sem2, sem3, sem4, sem5, sem6, sem7)

    for p in range(_NBUF - 1):
        pltpu.async_copy(x_hbm.at[row0 + p], buf.at[pl.ds(p * _N, _N)],
                         sems[p])

    def row_step(r, carry):
        par = lax.rem(r, _NBUF)

        @pl.when(r < _RPW_H - (_NBUF - 1))
        def _start_next():
            src = x_hbm.at[row0 + r + (_NBUF - 1)]
            npar = lax.rem(r + (_NBUF - 1), _NBUF)
            for p in range(_NBUF):
                @pl.when(npar == p)
                def _(p=p):
                    pltpu.async_copy(src, buf.at[pl.ds(p * _N, _N)],
                                     sems[p])

        for p in range(_NBUF):
            @pl.when(par == p)
            def _(p=p):
                pltpu.make_async_copy(x_hbm.at[row0],
                                      buf.at[pl.ds(p * _N, _N)],
                                      sems[p]).wait()

        rb = par * _N

        def a_body(g, acc):
            base = rb + g * _GSZ
            m = buf[pl.ds(base, _L)]
            for j in range(1, _L):
                m = jnp.maximum(m, buf[pl.ds(base + j * _L, _L)])
            bm[pl.ds(g * _L, _L)] = m
            return jnp.maximum(acc, m)

        lmax = lax.fori_loop(0, _NG, a_body, neg)

        lsort, _ = plsc.sort_key_val(lmax, lmax, descending=True)
        thresh = lsort[_K - 1]

        cnt = jnp.int32(0)
        for g in range(_NG):
            m = bm[pl.ds(g * _L, _L)]
            msk = m >= thresh
            ids = iota + g * _L
            plsc.store_compressed(cand.at[pl.ds(cnt, _L)], ids, mask=msk)
            pc = plsc.all_reduce_population_count(msk)
            cnt = cnt + pc[0]

        def d_cond(st):
            return st[0] * _L < cnt

        def d_body(st):
            c = st[0]
            ts = list(st[1:])
            off = c * _L
            lanes_ok = (iota + off) < cnt
            ids = jnp.where(lanes_ok, cand[pl.ds(off, _L)], 0)
            bvec = rb + jnp.right_shift(ids, 4) * _GSZ + \
                jnp.bitwise_and(ids, _L - 1)
            for j in range(_L):
                v = plsc.load_gather(buf, [bvec + j * _L], mask=lanes_ok)
                t = jnp.where(lanes_ok, v, neg)
                for i in range(_K):
                    hi = jnp.maximum(ts[i], t)
                    t = jnp.minimum(ts[i], t)
                    ts[i] = hi
            return (c + 1, *ts)

        st = lax.while_loop(d_cond, d_body,
                            (jnp.int32(0),) + (neg,) * _K)
        ts = list(st[1:])

        def msort(v):
            s, _ = plsc.sort_key_val(v, v, descending=True)
            return s

        def merge(a, b):
            return msort(jnp.maximum(a, lax.rev(b, (0,))))

        s = [msort(v) for v in ts]
        s = [merge(s[0], s[1]), merge(s[2], s[3]),
             merge(s[4], s[5]), merge(s[6], s[7])]
        top = merge(merge(s[0], s[1]), merge(s[2], s[3]))
        plsc.store_compressed(outb.at[pl.ds(r * _K, _L)], top,
                              mask=iota < _K)
        return carry

    lax.fori_loop(0, _RPW_H, row_step, 0)
    pltpu.sync_copy(outb.at[pl.ds(0, _RPW_H * _K)],
                    out_hbm.at[pl.ds(w * _RPW_H * _K, _RPW_H * _K)])


def _tc_body(x_ref, o_ref):
    xv = x_ref[0]
    rr, n = xv.shape
    col = lax.broadcasted_iota(jnp.int32, (rr, n), 1)
    outs = []
    for _ in range(_K):
        m = jnp.max(xv, axis=1, keepdims=True)
        eq = xv == m
        first = jnp.min(jnp.where(eq, col, n), axis=1, keepdims=True)
        xv = jnp.where(col == first, _NEG, xv)
        outs.append(m)
    o_ref[0] = jnp.concatenate(outs, axis=1)


def kernel(x):
    b, r, n = x.shape
    xf = x.reshape(b * r, n)

    tc_out = pl.pallas_call(
        _tc_body,
        grid=(_SPLIT,),
        in_specs=[pl.BlockSpec((1, r, n), lambda i: (i, 0, 0))],
        out_specs=pl.BlockSpec((1, r, _K), lambda i: (i, 0, 0)),
        out_shape=jax.ShapeDtypeStruct((_SPLIT, r, _K), jnp.float32),
    )(x).reshape(_SPLIT, r * _K)

    mesh = plsc.VectorSubcoreMesh(core_axis_name="c", subcore_axis_name="s",
                                  num_cores=2, num_subcores=16)
    run = pl.kernel(
        _sc_body,
        out_type=jax.ShapeDtypeStruct(((_B - _SPLIT) * _R * _K,),
                                      jnp.float32),
        mesh=mesh,
        scratch_types=[
            pltpu.VMEM((_NBUF * _N,), jnp.float32),
            pltpu.VMEM((_NG * _L,), jnp.float32),
            pltpu.VMEM((_NG * _L + 2 * _L,), jnp.int32),
            pltpu.VMEM((_RPW_H * _K + _L,), jnp.float32),
            pltpu.SemaphoreType.DMA,
            pltpu.SemaphoreType.DMA,
            pltpu.SemaphoreType.DMA,
            pltpu.SemaphoreType.DMA,
            pltpu.SemaphoreType.DMA,
            pltpu.SemaphoreType.DMA,
            pltpu.SemaphoreType.DMA,
            pltpu.SemaphoreType.DMA,
        ],
        compiler_params=pltpu.CompilerParams(needs_layout_passes=False),
    )
    sc_out = run(xf).reshape(_B - _SPLIT, _R * _K)
    return jnp.concatenate([tc_out, sc_out], axis=0)


# speedup vs baseline: 1.0727x; 1.0727x over previous
import functools

import jax
import jax.numpy as jnp
from jax import lax
from jax.experimental import pallas as pl
from jax.experimental.pallas import tpu as pltpu
from jax.experimental.pallas import tpu_sc as plsc

_K = 8
_L = 16
_N = 8192
_B = 32
_R = 32
_NW = 32
_RPW = (_B * _R) // _NW
_GSZ = _L * _L
_NG = _N // _GSZ
_NEG = float("-inf")


_NBUF = 8


_SPLIT = 8
_RPW_H = _R - _SPLIT


def _sc_body(x_hbm, out_hbm, buf, bm, cand, outb,
             sem0, sem1, sem2, sem3, sem4, sem5, sem6, sem7):
    w = lax.axis_index("s") * 2 + lax.axis_index("c")
    row0 = _SPLIT * _R + w * _RPW_H
    iota = lax.iota(jnp.int32, _L)
    neg = jnp.full((_L,), _NEG, jnp.float32)

    sems = (sem0, sem1, sem2, sem3, sem4, sem5, sem6, sem7)

    for p in range(_NBUF - 1):
        pltpu.async_copy(x_hbm.at[row0 + p], buf.at[pl.ds(p * _N, _N)],
                         sems[p])

    def row_step(r, carry):
        par = lax.rem(r, _NBUF)

        @pl.when(r < _RPW_H - (_NBUF - 1))
        def _start_next():
            src = x_hbm.at[row0 + r + (_NBUF - 1)]
            npar = lax.rem(r + (_NBUF - 1), _NBUF)
            for p in range(_NBUF):
                @pl.when(npar == p)
                def _(p=p):
                    pltpu.async_copy(src, buf.at[pl.ds(p * _N, _N)],
                                     sems[p])

        for p in range(_NBUF):
            @pl.when(par == p)
            def _(p=p):
                pltpu.make_async_copy(x_hbm.at[row0],
                                      buf.at[pl.ds(p * _N, _N)],
                                      sems[p]).wait()

        rb = par * _N

        def a_body(g, acc):
            base = rb + g * _GSZ
            m = buf[pl.ds(base, _L)]
            for j in range(1, _L):
                m = jnp.maximum(m, buf[pl.ds(base + j * _L, _L)])
            bm[pl.ds(g * _L, _L)] = m
            return jnp.maximum(acc, m)

        lmax = lax.fori_loop(0, _NG, a_body, neg)

        lsort, _ = plsc.sort_key_val(lmax, lmax, descending=True)
        thresh = lsort[_K - 1]

        cnt = jnp.int32(0)
        for g in range(_NG):
            m = bm[pl.ds(g * _L, _L)]
            msk = m >= thresh
            ids = iota + g * _L
            plsc.store_compressed(cand.at[pl.ds(cnt, _L)], ids, mask=msk)
            pc = plsc.all_reduce_population_count(msk)
            cnt = cnt + pc[0]

        def d_cond(st):
            return st[0] * _L < cnt

        def d_body(st):
            c = st[0]
            ts = list(st[1:])
            off = c * _L
            lanes_ok = (iota + off) < cnt
            ids = jnp.where(lanes_ok, cand[pl.ds(off, _L)], 0)
            bvec = rb + jnp.right_shift(ids, 4) * _GSZ + \
                jnp.bitwise_and(ids, _L - 1)
            for j in range(_L):
                v = plsc.load_gather(buf, [bvec + j * _L], mask=lanes_ok)
                t = jnp.where(lanes_ok, v, neg)
                for i in range(_K):
                    hi = jnp.maximum(ts[i], t)
                    t = jnp.minimum(ts[i], t)
                    ts[i] = hi
            return (c + 1, *ts)

        st = lax.while_loop(d_cond, d_body,
                            (jnp.int32(0),) + (neg,) * _K)
        ts = list(st[1:])

        def msort(v):
            s, _ = plsc.sort_key_val(v, v, descending=True)
            return s

        def merge(a, b):
            return msort(jnp.maximum(a, lax.rev(b, (0,))))

        s = [msort(v) for v in ts]
        s = [merge(s[0], s[1]), merge(s[2], s[3]),
             merge(s[4], s[5]), merge(s[6], s[7])]
        top = merge(merge(s[0], s[1]), merge(s[2], s[3]))
        plsc.store_compressed(outb.at[pl.ds(r * _K, _L)], top,
                              mask=iota < _K)
        return carry

    lax.fori_loop(0, _RPW_H, row_step, 0)
    pltpu.sync_copy(outb.at[pl.ds(0, _RPW_H * _K)],
                    out_hbm.at[pl.ds(w * _RPW_H * _K, _RPW_H * _K)])


def _tc_body(x_ref, o_ref):
    xv = x_ref[0]
    rr, n = xv.shape
    col = lax.broadcasted_iota(jnp.int32, (rr, n), 1)
    outs = []
    for _ in range(_K):
        m = jnp.max(xv, axis=1, keepdims=True)
        eq = xv == m
        first = jnp.min(jnp.where(eq, col, n), axis=1, keepdims=True)
        xv = jnp.where(col == first, _NEG, xv)
        outs.append(m)
    o_ref[0] = jnp.concatenate(outs, axis=1)


def kernel(x):
    b, r, n = x.shape
    xf = x.reshape(b * r, n)

    tc_out = pl.pallas_call(
        _tc_body,
        grid=(_SPLIT,),
        in_specs=[pl.BlockSpec((1, r, n), lambda i: (i, 0, 0))],
        out_specs=pl.BlockSpec((1, r, _K), lambda i: (i, 0, 0)),
        out_shape=jax.ShapeDtypeStruct((_SPLIT, r, _K), jnp.float32),
    )(x).reshape(_SPLIT, r * _K)

    mesh = plsc.VectorSubcoreMesh(core_axis_name="c", subcore_axis_name="s",
                                  num_cores=2, num_subcores=16)
    run = pl.kernel(
        _sc_body,
        out_type=jax.ShapeDtypeStruct(((_B - _SPLIT) * _R * _K,),
                                      jnp.float32),
        mesh=mesh,
        scratch_types=[
            pltpu.VMEM((_NBUF * _N,), jnp.float32),
            pltpu.VMEM((_NG * _L,), jnp.float32),
            pltpu.VMEM((_NG * _L + 2 * _L,), jnp.int32),
            pltpu.VMEM((_RPW_H * _K + _L,), jnp.float32),
            pltpu.SemaphoreType.DMA,
            pltpu.SemaphoreType.DMA,
            pltpu.SemaphoreType.DMA,
            pltpu.SemaphoreType.DMA,
            pltpu.SemaphoreType.DMA,
            pltpu.SemaphoreType.DMA,
            pltpu.SemaphoreType.DMA,
            pltpu.SemaphoreType.DMA,
        ],
        compiler_params=pltpu.CompilerParams(needs_layout_passes=False),
    )
    sc_out = run(xf).reshape(_B - _SPLIT, _R * _K)
    return jnp.concatenate([tc_out, sc_out], axis=0)
